# Initial kernel scaffold; baseline (speedup 1.0000x reference)
import jax
import jax.numpy as jnp
from jax.experimental import pallas as pl


def kernel(scores, bbox_deltas, im_info):
    raise NotImplementedError("write your pallas kernel here")


# speedup vs baseline: 87.9558x; 87.9558x over previous
import functools

import numpy as np
import jax
import jax.numpy as jnp
from jax import lax
from jax.experimental import pallas as pl
from jax.experimental.pallas import tpu as pltpu

_FPN_SCALES = [8, 16, 32, 64, 128]
_FPN_STRIDES = [4, 8, 16, 32, 64]
_ANCHOR_RATIOS = np.array([0.5, 1.0, 2.0], dtype=np.float64)
_FEAT_SHAPES = [(128, 128), (64, 64), (32, 32), (16, 16), (8, 8)]
_PRE_NMS = 12000
_POST_NMS = 2000
_NMS_THRESH = 0.7

_NB = 94
_NPAD = _NB * 128


def _anchors_np():
    anchors = []
    for lvl in range(len(_FPN_SCALES)):
        scale, stride, shape = _FPN_SCALES[lvl], _FPN_STRIDES[lvl], _FEAT_SHAPES[lvl]
        scales_g, ratios_g = np.meshgrid(np.array([scale], dtype=np.float64), _ANCHOR_RATIOS)
        scales_f = scales_g.flatten()
        ratios_f = ratios_g.flatten()
        heights = scales_f / np.sqrt(ratios_f)
        widths = scales_f * np.sqrt(ratios_f)
        shifts_y = np.arange(0, shape[0], 1) * stride
        shifts_x = np.arange(0, shape[1], 1) * stride
        shifts_x, shifts_y = np.meshgrid(shifts_x, shifts_y)
        box_widths, box_centers_x = np.meshgrid(widths, shifts_x)
        box_heights, box_centers_y = np.meshgrid(heights, shifts_y)
        box_centers = np.stack([box_centers_x, box_centers_y], axis=2).reshape(-1, 2)
        box_sizes = np.stack([box_widths, box_heights], axis=2).reshape(-1, 2)
        anchors.append(np.concatenate(
            [box_centers - 0.5 * box_sizes, box_centers + 0.5 * box_sizes], axis=1))
    return np.concatenate(anchors, axis=0).astype(np.float32)


def _nms_body(anc_ref, del_ref, clip_ref, out_ref,
              x1_ref, y1_ref, x2_ref, y2_ref, ar_ref, alive_ref, cnt_ref):
    f32 = jnp.float32
    anc = anc_ref[0]
    dlt = del_ref[0]
    ax1, ay1, ax2, ay2 = anc[0], anc[1], anc[2], anc[3]
    dx, dy, dw, dh = dlt[0], dlt[1], dlt[2], dlt[3]

    aw = ax2 - ax1 + 1.0
    ah = ay2 - ay1 + 1.0
    acx = ax1 + 0.5 * aw
    acy = ay1 + 0.5 * ah
    pcx = dx * aw + acx
    pcy = dy * ah + acy
    pw = jnp.exp(dw) * aw
    ph = jnp.exp(dh) * ah
    wm = clip_ref[0, 0:1, :]
    hm = clip_ref[0, 1:2, :]
    x1 = jnp.clip(pcx - 0.5 * pw, 0.0, wm)
    y1 = jnp.clip(pcy - 0.5 * ph, 0.0, hm)
    x2 = jnp.clip(pcx + 0.5 * pw, 0.0, wm)
    y2 = jnp.clip(pcy + 0.5 * ph, 0.0, hm)
    areas = (x2 - x1 + 1.0) * (y2 - y1 + 1.0)
    x1_ref[:] = x1
    y1_ref[:] = y1
    x2_ref[:] = x2
    y2_ref[:] = y2
    ar_ref[:] = areas

    ii = lax.broadcasted_iota(jnp.int32, (128, 128), 0)
    jj = lax.broadcasted_iota(jnp.int32, (128, 128), 1)
    ident = (ii == jj).astype(f32)

    def tcol(rows):
        return lax.dot_general(ident, rows, (((1,), (1,)), ((), ())),
                               preferred_element_type=f32)

    tri = (ii < jj).astype(f32)

    def row_of(ref, r):
        return ref[pl.ds(r, 1), :]

    lane = lax.broadcasted_iota(jnp.int32, (_NB, 128), 1)
    rowi = lax.broadcasted_iota(jnp.int32, (_NB, 128), 0)
    gidx = (rowi * 128 + lane).astype(f32)
    alive_ref[:] = jnp.where(gidx < float(_PRE_NMS), 1.0, 0.0)
    cnt_ref[0] = 0

    def block_body(b0, carry):
        @pl.when(cnt_ref[0] < _POST_NMS)
        def _():
            x1r = row_of(x1_ref, b0)
            y1r = row_of(y1_ref, b0)
            x2r = row_of(x2_ref, b0)
            y2r = row_of(y2_ref, b0)
            arr = row_of(ar_ref, b0)
            x1c = tcol(x1r)
            y1c = tcol(y1r)
            x2c = tcol(x2r)
            y2c = tcol(y2r)
            arc = tcol(arr)

            xx1 = jnp.maximum(x1c, x1r)
            yy1 = jnp.maximum(y1c, y1r)
            xx2 = jnp.minimum(x2c, x2r)
            yy2 = jnp.minimum(y2c, y2r)
            w = jnp.maximum(0.0, xx2 - xx1 + 1.0)
            h = jnp.maximum(0.0, yy2 - yy1 + 1.0)
            inter = w * h
            iou = inter / (arc + arr - inter)
            omat = jnp.where(iou > _NMS_THRESH, tri, 0.0)

            a0 = alive_ref[pl.ds(b0, 1), :]

            def wcond(st):
                return st[1]

            def wbody(st):
                a, _ = st
                supp = jnp.dot(a, omat, preferred_element_type=f32)
                na = jnp.where(supp > 0.5, 0.0, a0)
                return na, jnp.any(na != a)

            a_fin, _ = lax.while_loop(wcond, wbody, (a0, True))
            alive_ref[pl.ds(b0, 1), :] = a_fin
            nkept = jnp.sum(a_fin)
            cnt_ref[0] = cnt_ref[0] + nkept.astype(jnp.int32)

            @pl.when(nkept > 0.0)
            def _():
                def srow(c, _c):
                    tx1 = row_of(x1_ref, c)
                    ty1 = row_of(y1_ref, c)
                    tx2 = row_of(x2_ref, c)
                    ty2 = row_of(y2_ref, c)
                    tar = row_of(ar_ref, c)
                    sxx1 = jnp.maximum(x1c, tx1)
                    syy1 = jnp.maximum(y1c, ty1)
                    sxx2 = jnp.minimum(x2c, tx2)
                    syy2 = jnp.minimum(y2c, ty2)
                    sw = jnp.maximum(0.0, sxx2 - sxx1 + 1.0)
                    sh = jnp.maximum(0.0, syy2 - syy1 + 1.0)
                    sint = sw * sh
                    siou = sint / (arc + tar - sint)
                    smat = jnp.where(siou > _NMS_THRESH, 1.0, 0.0)
                    cntv = jnp.dot(a_fin, smat, preferred_element_type=f32)
                    trow = alive_ref[pl.ds(c, 1), :]
                    alive_ref[pl.ds(c, 1), :] = jnp.where(cntv > 0.5, 0.0, trow)
                    return _c

                lax.fori_loop(b0 + 1, _NB, srow, 0)

        return carry

    lax.fori_loop(0, _NB, block_body, 0)

    alive = alive_ref[:]
    tinc = (ii <= jj).astype(f32)
    csum_inc = jnp.dot(alive, tinc, preferred_element_type=f32)
    rs = csum_inc[:, 127:128]
    ri = lax.broadcasted_iota(jnp.int32, (_NB, _NB), 0)
    ci = lax.broadcasted_iota(jnp.int32, (_NB, _NB), 1)
    ustrict = (ci < ri).astype(f32)
    rowoff = jnp.dot(ustrict, rs, preferred_element_type=f32)
    pos = rowoff + csum_inc - alive
    posm = jnp.where(alive > 0.5, pos, 1e9)
    alive_ref[:] = posm

    srange = lax.broadcasted_iota(jnp.int32, (_POST_NMS, 1), 0).astype(f32)

    def obody(c, acc):
        prow = alive_ref[pl.ds(c, 1), :]
        sel = (prow == srange).astype(f32)
        rstack = jnp.concatenate(
            [row_of(x1_ref, c), row_of(y1_ref, c), row_of(x2_ref, c), row_of(y2_ref, c)], axis=0)
        boxc = tcol(rstack)
        return acc + jnp.dot(sel, boxc, preferred_element_type=f32)

    acc = lax.fori_loop(0, _NB, obody, jnp.zeros((_POST_NMS, 4), f32))
    out_ref[0] = acc


def _run_nms(anc4, del4, clipb):
    B = anc4.shape[0]
    return pl.pallas_call(
        _nms_body,
        grid=(B,),
        in_specs=[
            pl.BlockSpec((1, 4, _NB, 128), lambda b: (b, 0, 0, 0)),
            pl.BlockSpec((1, 4, _NB, 128), lambda b: (b, 0, 0, 0)),
            pl.BlockSpec((1, 2, 128), lambda b: (b, 0, 0)),
        ],
        out_specs=pl.BlockSpec((1, _POST_NMS, 4), lambda b: (b, 0, 0)),
        out_shape=jax.ShapeDtypeStruct((B, _POST_NMS, 4), jnp.float32),
        scratch_shapes=[
            pltpu.VMEM((_NB, 128), jnp.float32),
            pltpu.VMEM((_NB, 128), jnp.float32),
            pltpu.VMEM((_NB, 128), jnp.float32),
            pltpu.VMEM((_NB, 128), jnp.float32),
            pltpu.VMEM((_NB, 128), jnp.float32),
            pltpu.VMEM((_NB, 128), jnp.float32),
            pltpu.SMEM((1,), jnp.int32),
        ],
    )(anc4, del4, clipb)


def kernel(scores, bbox_deltas, im_info):
    B = bbox_deltas.shape[0]
    anchors = jnp.asarray(_anchors_np())
    sc = scores[:, :, 1]

    _, oi = lax.top_k(sc, _PRE_NMS)
    anc_g = jnp.take(anchors, oi, axis=0)
    del_g = jnp.take_along_axis(bbox_deltas, oi[:, :, None], axis=1)

    pad = _NPAD - _PRE_NMS
    anc_p = jnp.pad(anc_g, ((0, 0), (0, pad), (0, 0)))
    del_p = jnp.pad(del_g, ((0, 0), (0, pad), (0, 0)))
    anc4 = anc_p.transpose(0, 2, 1).reshape(B, 4, _NB, 128)
    del4 = del_p.transpose(0, 2, 1).reshape(B, 4, _NB, 128)

    wm = jnp.broadcast_to((im_info[:, 1] - 1.0)[:, None], (B, 128))
    hm = jnp.broadcast_to((im_info[:, 0] - 1.0)[:, None], (B, 128))
    clipb = jnp.stack([wm, hm], axis=1)

    boxes = _run_nms(anc4, del4, clipb)

    bcol = jnp.broadcast_to(
        jnp.arange(B, dtype=jnp.float32)[:, None, None], (B, _POST_NMS, 1))
    return jnp.concatenate([bcol, boxes], axis=2)


# speedup vs baseline: 125.1013x; 1.4223x over previous
import functools

import numpy as np
import jax
import jax.numpy as jnp
from jax import lax
from jax.experimental import pallas as pl
from jax.experimental.pallas import tpu as pltpu

_FPN_SCALES = [8, 16, 32, 64, 128]
_FPN_STRIDES = [4, 8, 16, 32, 64]
_ANCHOR_RATIOS = np.array([0.5, 1.0, 2.0], dtype=np.float64)
_FEAT_SHAPES = [(128, 128), (64, 64), (32, 32), (16, 16), (8, 8)]
_PRE_NMS = 12000
_POST_NMS = 2000
_NMS_THRESH = 0.7

_NB = 94
_NPAD = _NB * 128


def _anchors_np():
    anchors = []
    for lvl in range(len(_FPN_SCALES)):
        scale, stride, shape = _FPN_SCALES[lvl], _FPN_STRIDES[lvl], _FEAT_SHAPES[lvl]
        scales_g, ratios_g = np.meshgrid(np.array([scale], dtype=np.float64), _ANCHOR_RATIOS)
        scales_f = scales_g.flatten()
        ratios_f = ratios_g.flatten()
        heights = scales_f / np.sqrt(ratios_f)
        widths = scales_f * np.sqrt(ratios_f)
        shifts_y = np.arange(0, shape[0], 1) * stride
        shifts_x = np.arange(0, shape[1], 1) * stride
        shifts_x, shifts_y = np.meshgrid(shifts_x, shifts_y)
        box_widths, box_centers_x = np.meshgrid(widths, shifts_x)
        box_heights, box_centers_y = np.meshgrid(heights, shifts_y)
        box_centers = np.stack([box_centers_x, box_centers_y], axis=2).reshape(-1, 2)
        box_sizes = np.stack([box_widths, box_heights], axis=2).reshape(-1, 2)
        anchors.append(np.concatenate(
            [box_centers - 0.5 * box_sizes, box_centers + 0.5 * box_sizes], axis=1))
    return np.concatenate(anchors, axis=0).astype(np.float32)


_roll = pltpu.roll


def _sort_body(key_ref, idx_ref, outi_ref):
    k = key_ref[:]
    v = idx_ref[:]
    R = k.shape[0]
    lane = lax.broadcasted_iota(jnp.int32, (R, 128), 1)
    rowl = lax.broadcasted_iota(jnp.int32, (R, 128), 0) % 512

    def partner(arr, j):
        if j < 128:
            lo = (lane & j) == 0
            return jnp.where(lo, _roll(arr, 128 - j, 1), _roll(arr, j, 1))
        m = j // 128
        a = arr.reshape(R // (2 * m), 2, m, 128)
        sw = jnp.concatenate([a[:, 1:2], a[:, 0:1]], axis=1)
        return sw.reshape(R, 128)

    for lev in range(1, 17):
        kbit = 1 << lev
        for j_exp in range(lev - 1, -1, -1):
            j = 1 << j_exp
            pk = partner(k, j)
            pv = partner(v, j)
            low = ((lane & j) == 0) if j < 128 else ((rowl & (j // 128)) == 0)
            if kbit < 128:
                desc = (lane & kbit) == 0
            else:
                desc = (rowl & (kbit // 128)) == 0
            better = (k > pk) | ((k == pk) & (v < pv))
            takem = (desc == low) == better
            k = jnp.where(takem, k, pk)
            v = jnp.where(takem, v, pv)
    outi_ref[:] = v


def _run_sort(keys, idx):
    R = keys.shape[0]
    return pl.pallas_call(
        _sort_body,
        out_shape=jax.ShapeDtypeStruct((R, 128), jnp.int32),
    )(keys, idx)


def _nms_body(anc_ref, del_ref, clip_ref, out_ref,
              x1_ref, y1_ref, x2_ref, y2_ref, ar_ref, alive_ref, cnt_ref):
    f32 = jnp.float32
    anc = anc_ref[0]
    dlt = del_ref[0]
    ax1, ay1, ax2, ay2 = anc[0], anc[1], anc[2], anc[3]
    dx, dy, dw, dh = dlt[0], dlt[1], dlt[2], dlt[3]

    aw = ax2 - ax1 + 1.0
    ah = ay2 - ay1 + 1.0
    acx = ax1 + 0.5 * aw
    acy = ay1 + 0.5 * ah
    pcx = dx * aw + acx
    pcy = dy * ah + acy
    pw = jnp.exp(dw) * aw
    ph = jnp.exp(dh) * ah
    wm = clip_ref[0, 0:1, :]
    hm = clip_ref[0, 1:2, :]
    x1 = jnp.clip(pcx - 0.5 * pw, 0.0, wm)
    y1 = jnp.clip(pcy - 0.5 * ph, 0.0, hm)
    x2 = jnp.clip(pcx + 0.5 * pw, 0.0, wm)
    y2 = jnp.clip(pcy + 0.5 * ph, 0.0, hm)
    areas = (x2 - x1 + 1.0) * (y2 - y1 + 1.0)
    x1_ref[:] = x1
    y1_ref[:] = y1
    x2_ref[:] = x2
    y2_ref[:] = y2
    ar_ref[:] = areas

    ii = lax.broadcasted_iota(jnp.int32, (128, 128), 0)
    jj = lax.broadcasted_iota(jnp.int32, (128, 128), 1)
    ident = (ii == jj).astype(f32)

    def tcol(rows):
        return lax.dot_general(ident, rows, (((1,), (1,)), ((), ())),
                               preferred_element_type=f32)

    tri = (ii < jj).astype(f32)

    def row_of(ref, r):
        return ref[pl.ds(r, 1), :]

    lane = lax.broadcasted_iota(jnp.int32, (_NB, 128), 1)
    rowi = lax.broadcasted_iota(jnp.int32, (_NB, 128), 0)
    gidx = (rowi * 128 + lane).astype(f32)
    alive_ref[:] = jnp.where(gidx < float(_PRE_NMS), 1.0, 0.0)
    cnt_ref[0] = 0

    def block_body(b0, carry):
        @pl.when(cnt_ref[0] < _POST_NMS)
        def _():
            x1r = row_of(x1_ref, b0)
            y1r = row_of(y1_ref, b0)
            x2r = row_of(x2_ref, b0)
            y2r = row_of(y2_ref, b0)
            arr = row_of(ar_ref, b0)
            x1c = tcol(x1r)
            y1c = tcol(y1r)
            x2c = tcol(x2r)
            y2c = tcol(y2r)
            arc = tcol(arr)

            xx1 = jnp.maximum(x1c, x1r)
            yy1 = jnp.maximum(y1c, y1r)
            xx2 = jnp.minimum(x2c, x2r)
            yy2 = jnp.minimum(y2c, y2r)
            w = jnp.maximum(0.0, xx2 - xx1 + 1.0)
            h = jnp.maximum(0.0, yy2 - yy1 + 1.0)
            inter = w * h
            iou = inter / (arc + arr - inter)
            omat = jnp.where(iou > _NMS_THRESH, tri, 0.0)

            a0 = alive_ref[pl.ds(b0, 1), :]

            def wcond(st):
                return st[1]

            def wbody(st):
                a, _ = st
                supp = jnp.dot(a, omat, preferred_element_type=f32)
                na = jnp.where(supp > 0.5, 0.0, a0)
                return na, jnp.any(na != a)

            a_fin, _ = lax.while_loop(wcond, wbody, (a0, True))
            alive_ref[pl.ds(b0, 1), :] = a_fin
            nkept = jnp.sum(a_fin)
            cnt_ref[0] = cnt_ref[0] + nkept.astype(jnp.int32)

            @pl.when(nkept > 0.0)
            def _():
                def srow(c, _c):
                    tx1 = row_of(x1_ref, c)
                    ty1 = row_of(y1_ref, c)
                    tx2 = row_of(x2_ref, c)
                    ty2 = row_of(y2_ref, c)
                    tar = row_of(ar_ref, c)
                    sxx1 = jnp.maximum(x1c, tx1)
                    syy1 = jnp.maximum(y1c, ty1)
                    sxx2 = jnp.minimum(x2c, tx2)
                    syy2 = jnp.minimum(y2c, ty2)
                    sw = jnp.maximum(0.0, sxx2 - sxx1 + 1.0)
                    sh = jnp.maximum(0.0, syy2 - syy1 + 1.0)
                    sint = sw * sh
                    siou = sint / (arc + tar - sint)
                    smat = jnp.where(siou > _NMS_THRESH, 1.0, 0.0)
                    cntv = jnp.dot(a_fin, smat, preferred_element_type=f32)
                    trow = alive_ref[pl.ds(c, 1), :]
                    alive_ref[pl.ds(c, 1), :] = jnp.where(cntv > 0.5, 0.0, trow)
                    return _c

                lax.fori_loop(b0 + 1, _NB, srow, 0)

        return carry

    lax.fori_loop(0, _NB, block_body, 0)

    alive = alive_ref[:]
    tinc = (ii <= jj).astype(f32)
    csum_inc = jnp.dot(alive, tinc, preferred_element_type=f32)
    rs = csum_inc[:, 127:128]
    ri = lax.broadcasted_iota(jnp.int32, (_NB, _NB), 0)
    ci = lax.broadcasted_iota(jnp.int32, (_NB, _NB), 1)
    ustrict = (ci < ri).astype(f32)
    rowoff = jnp.dot(ustrict, rs, preferred_element_type=f32)
    pos = rowoff + csum_inc - alive
    posm = jnp.where(alive > 0.5, pos, 1e9)
    alive_ref[:] = posm

    srange = lax.broadcasted_iota(jnp.int32, (_POST_NMS, 1), 0).astype(f32)

    def obody(c, acc):
        prow = alive_ref[pl.ds(c, 1), :]
        sel = (prow == srange).astype(f32)
        rstack = jnp.concatenate(
            [row_of(x1_ref, c), row_of(y1_ref, c), row_of(x2_ref, c), row_of(y2_ref, c)], axis=0)
        boxc = tcol(rstack)
        return acc + jnp.dot(sel, boxc, preferred_element_type=f32)

    acc = lax.fori_loop(0, _NB, obody, jnp.zeros((_POST_NMS, 4), f32))
    out_ref[0] = acc


def _run_nms(anc4, del4, clipb):
    B = anc4.shape[0]
    return pl.pallas_call(
        _nms_body,
        grid=(B,),
        in_specs=[
            pl.BlockSpec((1, 4, _NB, 128), lambda b: (b, 0, 0, 0)),
            pl.BlockSpec((1, 4, _NB, 128), lambda b: (b, 0, 0, 0)),
            pl.BlockSpec((1, 2, 128), lambda b: (b, 0, 0)),
        ],
        out_specs=pl.BlockSpec((1, _POST_NMS, 4), lambda b: (b, 0, 0)),
        out_shape=jax.ShapeDtypeStruct((B, _POST_NMS, 4), jnp.float32),
        scratch_shapes=[
            pltpu.VMEM((_NB, 128), jnp.float32),
            pltpu.VMEM((_NB, 128), jnp.float32),
            pltpu.VMEM((_NB, 128), jnp.float32),
            pltpu.VMEM((_NB, 128), jnp.float32),
            pltpu.VMEM((_NB, 128), jnp.float32),
            pltpu.VMEM((_NB, 128), jnp.float32),
            pltpu.SMEM((1,), jnp.int32),
        ],
    )(anc4, del4, clipb)


def kernel(scores, bbox_deltas, im_info):
    B = bbox_deltas.shape[0]
    anchors = jnp.asarray(_anchors_np())
    sc = scores[:, :, 1]

    kk = lax.bitcast_convert_type(sc, jnp.int32)
    kk = jnp.pad(kk, ((0, 0), (0, 65536 - kk.shape[1])), constant_values=-1)
    ii = jnp.broadcast_to(jnp.arange(65536, dtype=jnp.int32), (B, 65536))
    sorted_idx = _run_sort(kk.reshape(B * 512, 128), ii.reshape(B * 512, 128))
    oi = sorted_idx.reshape(B, 65536)[:, :_NPAD]

    anc_p = jnp.take(anchors, oi, axis=0)
    del_p = jnp.take_along_axis(bbox_deltas, oi[:, :, None], axis=1,
                                mode="promise_in_bounds")
    anc4 = anc_p.transpose(0, 2, 1).reshape(B, 4, _NB, 128)
    del4 = del_p.transpose(0, 2, 1).reshape(B, 4, _NB, 128)

    wm = jnp.broadcast_to((im_info[:, 1] - 1.0)[:, None], (B, 128))
    hm = jnp.broadcast_to((im_info[:, 0] - 1.0)[:, None], (B, 128))
    clipb = jnp.stack([wm, hm], axis=1)

    boxes = _run_nms(anc4, del4, clipb)

    bcol = jnp.broadcast_to(
        jnp.arange(B, dtype=jnp.float32)[:, None, None], (B, _POST_NMS, 1))
    return jnp.concatenate([bcol, boxes], axis=2)


# speedup vs baseline: 134.3036x; 1.0736x over previous
import functools

import numpy as np
import jax
import jax.numpy as jnp
from jax import lax
from jax.experimental import pallas as pl
from jax.experimental.pallas import tpu as pltpu
from jax.experimental.pallas import tpu_sc as plsc

_FPN_SCALES = [8, 16, 32, 64, 128]
_FPN_STRIDES = [4, 8, 16, 32, 64]
_ANCHOR_RATIOS = np.array([0.5, 1.0, 2.0], dtype=np.float64)
_FEAT_SHAPES = [(128, 128), (64, 64), (32, 32), (16, 16), (8, 8)]
_PRE_NMS = 12000
_POST_NMS = 2000
_NMS_THRESH = 0.7

_NB = 94
_NPAD = _NB * 128


def _anchors_np():
    anchors = []
    for lvl in range(len(_FPN_SCALES)):
        scale, stride, shape = _FPN_SCALES[lvl], _FPN_STRIDES[lvl], _FEAT_SHAPES[lvl]
        scales_g, ratios_g = np.meshgrid(np.array([scale], dtype=np.float64), _ANCHOR_RATIOS)
        scales_f = scales_g.flatten()
        ratios_f = ratios_g.flatten()
        heights = scales_f / np.sqrt(ratios_f)
        widths = scales_f * np.sqrt(ratios_f)
        shifts_y = np.arange(0, shape[0], 1) * stride
        shifts_x = np.arange(0, shape[1], 1) * stride
        shifts_x, shifts_y = np.meshgrid(shifts_x, shifts_y)
        box_widths, box_centers_x = np.meshgrid(widths, shifts_x)
        box_heights, box_centers_y = np.meshgrid(heights, shifts_y)
        box_centers = np.stack([box_centers_x, box_centers_y], axis=2).reshape(-1, 2)
        box_sizes = np.stack([box_widths, box_heights], axis=2).reshape(-1, 2)
        anchors.append(np.concatenate(
            [box_centers - 0.5 * box_sizes, box_centers + 0.5 * box_sizes], axis=1))
    return np.concatenate(anchors, axis=0).astype(np.float32)


_roll = pltpu.roll

_GW = 32
_GCH = 6
_GPAD = 12288


def _make_sc_gather():
    mesh = plsc.VectorSubcoreMesh(core_axis_name="c", subcore_axis_name="s")

    @functools.partial(
        pl.kernel, mesh=mesh,
        out_type=jax.ShapeDtypeStruct((8, _GW, _GCH, 128), jnp.float32),
        scratch_types=[
            pltpu.VMEM((_GCH, 128), jnp.int32),
            pltpu.VMEM((_GCH, 128), jnp.int32),
            pltpu.VMEM((_GCH, 128), jnp.float32),
            pltpu.SemaphoreType.DMA,
        ],
    )
    def gk(aidx_hbm, didx_hbm, t0, t1, t2, t3, t4, t5, t6, t7,
           out_hbm, aidx_v, didx_v, buf_v, sem):
        w = lax.axis_index("s") * 2 + lax.axis_index("c")
        pltpu.sync_copy(aidx_hbm.at[w], aidx_v)
        pltpu.sync_copy(didx_hbm.at[w], didx_v)
        for c, tbl in enumerate((t0, t1, t2, t3, t4, t5, t6, t7)):
            idx_v = aidx_v if c < 4 else didx_v
            cps = [pltpu.async_copy(tbl.at[idx_v.at[j]], buf_v.at[j], sem)
                   for j in range(_GCH)]
            for cp in cps:
                cp.wait()
            pltpu.sync_copy(buf_v, out_hbm.at[c, w])

    return gk


def _sc_gather(oi, anchors, bbox_deltas):
    B, n = oi.shape
    aidx = jnp.pad(oi, ((0, 0), (0, _GPAD - n))).astype(jnp.int32)
    didx = aidx + (jnp.arange(B, dtype=jnp.int32) * 65472)[:, None]
    aidx = aidx.reshape(_GW, _GCH, 128)
    didx = didx.reshape(_GW, _GCH, 128)
    tables = [anchors[:, c] for c in range(4)]
    tables += [bbox_deltas[:, :, c].reshape(-1) for c in range(4)]
    g = _make_sc_gather()(aidx, didx, *tables)
    g = g.reshape(8, B, _GPAD // 128, 128)[:, :, :_NB, :]
    return g[0:4].transpose(1, 0, 2, 3), g[4:8].transpose(1, 0, 2, 3)


def _sort_body(key_ref, idx_ref, outi_ref):
    k = key_ref[:]
    v = idx_ref[:]
    R = k.shape[0]
    lane = lax.broadcasted_iota(jnp.int32, (R, 128), 1)
    rowl = lax.broadcasted_iota(jnp.int32, (R, 128), 0) % 512

    def partner(arr, j):
        if j < 128:
            lo = (lane & j) == 0
            return jnp.where(lo, _roll(arr, 128 - j, 1), _roll(arr, j, 1))
        m = j // 128
        a = arr.reshape(R // (2 * m), 2, m, 128)
        sw = jnp.concatenate([a[:, 1:2], a[:, 0:1]], axis=1)
        return sw.reshape(R, 128)

    for lev in range(1, 17):
        kbit = 1 << lev
        for j_exp in range(lev - 1, -1, -1):
            j = 1 << j_exp
            pk = partner(k, j)
            pv = partner(v, j)
            low = ((lane & j) == 0) if j < 128 else ((rowl & (j // 128)) == 0)
            if kbit < 128:
                desc = (lane & kbit) == 0
            else:
                desc = (rowl & (kbit // 128)) == 0
            better = (k > pk) | ((k == pk) & (v < pv))
            takem = (desc == low) == better
            k = jnp.where(takem, k, pk)
            v = jnp.where(takem, v, pv)
    outi_ref[:] = v


def _run_sort(keys, idx):
    R = keys.shape[0]
    return pl.pallas_call(
        _sort_body,
        out_shape=jax.ShapeDtypeStruct((R, 128), jnp.int32),
    )(keys, idx)


def _nms_body(anc_ref, del_ref, clip_ref, out_ref,
              x1_ref, y1_ref, x2_ref, y2_ref, ar_ref, alive_ref, cnt_ref):
    f32 = jnp.float32
    anc = anc_ref[0]
    dlt = del_ref[0]
    ax1, ay1, ax2, ay2 = anc[0], anc[1], anc[2], anc[3]
    dx, dy, dw, dh = dlt[0], dlt[1], dlt[2], dlt[3]

    aw = ax2 - ax1 + 1.0
    ah = ay2 - ay1 + 1.0
    acx = ax1 + 0.5 * aw
    acy = ay1 + 0.5 * ah
    pcx = dx * aw + acx
    pcy = dy * ah + acy
    pw = jnp.exp(dw) * aw
    ph = jnp.exp(dh) * ah
    wm = clip_ref[0, 0:1, :]
    hm = clip_ref[0, 1:2, :]
    x1 = jnp.clip(pcx - 0.5 * pw, 0.0, wm)
    y1 = jnp.clip(pcy - 0.5 * ph, 0.0, hm)
    x2 = jnp.clip(pcx + 0.5 * pw, 0.0, wm)
    y2 = jnp.clip(pcy + 0.5 * ph, 0.0, hm)
    areas = (x2 - x1 + 1.0) * (y2 - y1 + 1.0)
    x1_ref[:] = x1
    y1_ref[:] = y1
    x2_ref[:] = x2
    y2_ref[:] = y2
    ar_ref[:] = areas

    ii = lax.broadcasted_iota(jnp.int32, (128, 128), 0)
    jj = lax.broadcasted_iota(jnp.int32, (128, 128), 1)
    ident = (ii == jj).astype(f32)

    def tcol(rows):
        return lax.dot_general(ident, rows, (((1,), (1,)), ((), ())),
                               preferred_element_type=f32)

    tri = (ii < jj).astype(f32)

    def row_of(ref, r):
        return ref[pl.ds(r, 1), :]

    lane = lax.broadcasted_iota(jnp.int32, (_NB, 128), 1)
    rowi = lax.broadcasted_iota(jnp.int32, (_NB, 128), 0)
    gidx = (rowi * 128 + lane).astype(f32)
    alive_ref[:] = jnp.where(gidx < float(_PRE_NMS), 1.0, 0.0)
    cnt_ref[0] = 0

    def block_body(b0, carry):
        @pl.when(cnt_ref[0] < _POST_NMS)
        def _():
            x1r = row_of(x1_ref, b0)
            y1r = row_of(y1_ref, b0)
            x2r = row_of(x2_ref, b0)
            y2r = row_of(y2_ref, b0)
            arr = row_of(ar_ref, b0)
            x1c = tcol(x1r)
            y1c = tcol(y1r)
            x2c = tcol(x2r)
            y2c = tcol(y2r)
            arc = tcol(arr)

            xx1 = jnp.maximum(x1c, x1r)
            yy1 = jnp.maximum(y1c, y1r)
            xx2 = jnp.minimum(x2c, x2r)
            yy2 = jnp.minimum(y2c, y2r)
            w = jnp.maximum(0.0, xx2 - xx1 + 1.0)
            h = jnp.maximum(0.0, yy2 - yy1 + 1.0)
            inter = w * h
            iou = inter / (arc + arr - inter)
            omat = jnp.where(iou > _NMS_THRESH, tri, 0.0)

            a0 = alive_ref[pl.ds(b0, 1), :]

            def wcond(st):
                return st[1]

            def wbody(st):
                a, _ = st
                supp = jnp.dot(a, omat, preferred_element_type=f32)
                na = jnp.where(supp > 0.5, 0.0, a0)
                return na, jnp.any(na != a)

            a_fin, _ = lax.while_loop(wcond, wbody, (a0, True))
            alive_ref[pl.ds(b0, 1), :] = a_fin
            nkept = jnp.sum(a_fin)
            cnt_ref[0] = cnt_ref[0] + nkept.astype(jnp.int32)

            @pl.when(nkept > 0.0)
            def _():
                def srow(c, _c):
                    tx1 = row_of(x1_ref, c)
                    ty1 = row_of(y1_ref, c)
                    tx2 = row_of(x2_ref, c)
                    ty2 = row_of(y2_ref, c)
                    tar = row_of(ar_ref, c)
                    sxx1 = jnp.maximum(x1c, tx1)
                    syy1 = jnp.maximum(y1c, ty1)
                    sxx2 = jnp.minimum(x2c, tx2)
                    syy2 = jnp.minimum(y2c, ty2)
                    sw = jnp.maximum(0.0, sxx2 - sxx1 + 1.0)
                    sh = jnp.maximum(0.0, syy2 - syy1 + 1.0)
                    sint = sw * sh
                    siou = sint / (arc + tar - sint)
                    smat = jnp.where(siou > _NMS_THRESH, 1.0, 0.0)
                    cntv = jnp.dot(a_fin, smat, preferred_element_type=f32)
                    trow = alive_ref[pl.ds(c, 1), :]
                    alive_ref[pl.ds(c, 1), :] = jnp.where(cntv > 0.5, 0.0, trow)
                    return _c

                lax.fori_loop(b0 + 1, _NB, srow, 0)

        return carry

    lax.fori_loop(0, _NB, block_body, 0)

    alive = alive_ref[:]
    tinc = (ii <= jj).astype(f32)
    csum_inc = jnp.dot(alive, tinc, preferred_element_type=f32)
    rs = csum_inc[:, 127:128]
    ri = lax.broadcasted_iota(jnp.int32, (_NB, _NB), 0)
    ci = lax.broadcasted_iota(jnp.int32, (_NB, _NB), 1)
    ustrict = (ci < ri).astype(f32)
    rowoff = jnp.dot(ustrict, rs, preferred_element_type=f32)
    pos = rowoff + csum_inc - alive
    posm = jnp.where(alive > 0.5, pos, 1e9)
    alive_ref[:] = posm

    srange = lax.broadcasted_iota(jnp.int32, (_POST_NMS, 1), 0).astype(f32)

    def obody(c, acc):
        prow = alive_ref[pl.ds(c, 1), :]
        sel = (prow == srange).astype(f32)
        rstack = jnp.concatenate(
            [row_of(x1_ref, c), row_of(y1_ref, c), row_of(x2_ref, c), row_of(y2_ref, c)], axis=0)
        boxc = tcol(rstack)
        return acc + jnp.dot(sel, boxc, preferred_element_type=f32)

    acc = lax.fori_loop(0, _NB, obody, jnp.zeros((_POST_NMS, 4), f32))
    out_ref[0] = acc


def _run_nms(anc4, del4, clipb):
    B = anc4.shape[0]
    return pl.pallas_call(
        _nms_body,
        grid=(B,),
        in_specs=[
            pl.BlockSpec((1, 4, _NB, 128), lambda b: (b, 0, 0, 0)),
            pl.BlockSpec((1, 4, _NB, 128), lambda b: (b, 0, 0, 0)),
            pl.BlockSpec((1, 2, 128), lambda b: (b, 0, 0)),
        ],
        out_specs=pl.BlockSpec((1, _POST_NMS, 4), lambda b: (b, 0, 0)),
        out_shape=jax.ShapeDtypeStruct((B, _POST_NMS, 4), jnp.float32),
        scratch_shapes=[
            pltpu.VMEM((_NB, 128), jnp.float32),
            pltpu.VMEM((_NB, 128), jnp.float32),
            pltpu.VMEM((_NB, 128), jnp.float32),
            pltpu.VMEM((_NB, 128), jnp.float32),
            pltpu.VMEM((_NB, 128), jnp.float32),
            pltpu.VMEM((_NB, 128), jnp.float32),
            pltpu.SMEM((1,), jnp.int32),
        ],
    )(anc4, del4, clipb)


def kernel(scores, bbox_deltas, im_info):
    B = bbox_deltas.shape[0]
    anchors = jnp.asarray(_anchors_np())
    sc = scores[:, :, 1]

    kk = lax.bitcast_convert_type(sc, jnp.int32)
    kk = jnp.pad(kk, ((0, 0), (0, 65536 - kk.shape[1])), constant_values=-1)
    ii = jnp.broadcast_to(jnp.arange(65536, dtype=jnp.int32), (B, 65536))
    sorted_idx = _run_sort(kk.reshape(B * 512, 128), ii.reshape(B * 512, 128))
    oi = sorted_idx.reshape(B, 65536)[:, :_NPAD]

    anc4, del4 = _sc_gather(oi, anchors, bbox_deltas)

    wm = jnp.broadcast_to((im_info[:, 1] - 1.0)[:, None], (B, 128))
    hm = jnp.broadcast_to((im_info[:, 0] - 1.0)[:, None], (B, 128))
    clipb = jnp.stack([wm, hm], axis=1)

    boxes = _run_nms(anc4, del4, clipb)

    bcol = jnp.broadcast_to(
        jnp.arange(B, dtype=jnp.float32)[:, None, None], (B, _POST_NMS, 1))
    return jnp.concatenate([bcol, boxes], axis=2)


# speedup vs baseline: 230.8775x; 1.7191x over previous
import functools

import numpy as np
import jax
import jax.numpy as jnp
from jax import lax
from jax.experimental import pallas as pl
from jax.experimental.pallas import tpu as pltpu
from jax.experimental.pallas import tpu_sc as plsc

_FPN_SCALES = [8, 16, 32, 64, 128]
_FPN_STRIDES = [4, 8, 16, 32, 64]
_ANCHOR_RATIOS = np.array([0.5, 1.0, 2.0], dtype=np.float64)
_FEAT_SHAPES = [(128, 128), (64, 64), (32, 32), (16, 16), (8, 8)]
_PRE_NMS = 12000
_POST_NMS = 2000
_NMS_THRESH = 0.7

_NB = 94
_NPAD = _NB * 128


def _anchors_np():
    anchors = []
    for lvl in range(len(_FPN_SCALES)):
        scale, stride, shape = _FPN_SCALES[lvl], _FPN_STRIDES[lvl], _FEAT_SHAPES[lvl]
        scales_g, ratios_g = np.meshgrid(np.array([scale], dtype=np.float64), _ANCHOR_RATIOS)
        scales_f = scales_g.flatten()
        ratios_f = ratios_g.flatten()
        heights = scales_f / np.sqrt(ratios_f)
        widths = scales_f * np.sqrt(ratios_f)
        shifts_y = np.arange(0, shape[0], 1) * stride
        shifts_x = np.arange(0, shape[1], 1) * stride
        shifts_x, shifts_y = np.meshgrid(shifts_x, shifts_y)
        box_widths, box_centers_x = np.meshgrid(widths, shifts_x)
        box_heights, box_centers_y = np.meshgrid(heights, shifts_y)
        box_centers = np.stack([box_centers_x, box_centers_y], axis=2).reshape(-1, 2)
        box_sizes = np.stack([box_widths, box_heights], axis=2).reshape(-1, 2)
        anchors.append(np.concatenate(
            [box_centers - 0.5 * box_sizes, box_centers + 0.5 * box_sizes], axis=1))
    return np.concatenate(anchors, axis=0).astype(np.float32)


_roll = pltpu.roll

_GW = 32
_GCH = 6
_GPAD = 12288


def _make_sc_gather():
    mesh = plsc.VectorSubcoreMesh(core_axis_name="c", subcore_axis_name="s")

    @functools.partial(
        pl.kernel, mesh=mesh,
        out_type=jax.ShapeDtypeStruct((8, _GW, _GCH, 128), jnp.float32),
        scratch_types=[
            pltpu.VMEM((_GCH, 128), jnp.int32),
            pltpu.VMEM((_GCH, 128), jnp.int32),
            pltpu.VMEM((_GCH, 128), jnp.float32),
            pltpu.SemaphoreType.DMA,
        ],
    )
    def gk(aidx_hbm, didx_hbm, t0, t1, t2, t3, t4, t5, t6, t7,
           out_hbm, aidx_v, didx_v, buf_v, sem):
        w = lax.axis_index("s") * 2 + lax.axis_index("c")
        pltpu.sync_copy(aidx_hbm.at[w], aidx_v)
        pltpu.sync_copy(didx_hbm.at[w], didx_v)
        for c, tbl in enumerate((t0, t1, t2, t3, t4, t5, t6, t7)):
            idx_v = aidx_v if c < 4 else didx_v
            cps = [pltpu.async_copy(tbl.at[idx_v.at[j]], buf_v.at[j], sem)
                   for j in range(_GCH)]
            for cp in cps:
                cp.wait()
            pltpu.sync_copy(buf_v, out_hbm.at[c, w])

    return gk


def _sc_gather(oi, anchors, bbox_deltas):
    B, n = oi.shape
    aidx = jnp.pad(oi, ((0, 0), (0, _GPAD - n))).astype(jnp.int32)
    didx = aidx + (jnp.arange(B, dtype=jnp.int32) * 65472)[:, None]
    aidx = aidx.reshape(_GW, _GCH, 128)
    didx = didx.reshape(_GW, _GCH, 128)
    tables = [anchors[:, c] for c in range(4)]
    tables += [bbox_deltas[:, :, c].reshape(-1) for c in range(4)]
    g = _make_sc_gather()(aidx, didx, *tables)
    g = g.reshape(8, B, _GPAD // 128, 128)[:, :, :_NB, :]
    return g[0:4].transpose(1, 0, 2, 3), g[4:8].transpose(1, 0, 2, 3)


def _sort_body(key_ref, idx_ref, outi_ref):
    k = key_ref[:]
    v = idx_ref[:]
    R = k.shape[0]
    lane = lax.broadcasted_iota(jnp.int32, (R, 128), 1)
    rowl = lax.broadcasted_iota(jnp.int32, (R, 128), 0) % 512

    def partner(arr, j):
        if j < 128:
            lo = (lane & j) == 0
            return jnp.where(lo, _roll(arr, 128 - j, 1), _roll(arr, j, 1))
        m = j // 128
        a = arr.reshape(R // (2 * m), 2, m, 128)
        sw = jnp.concatenate([a[:, 1:2], a[:, 0:1]], axis=1)
        return sw.reshape(R, 128)

    for lev in range(1, 17):
        kbit = 1 << lev
        for j_exp in range(lev - 1, -1, -1):
            j = 1 << j_exp
            pk = partner(k, j)
            pv = partner(v, j)
            low = ((lane & j) == 0) if j < 128 else ((rowl & (j // 128)) == 0)
            if kbit < 128:
                desc = (lane & kbit) == 0
            else:
                desc = (rowl & (kbit // 128)) == 0
            better = (k > pk) | ((k == pk) & (v < pv))
            takem = (desc == low) == better
            k = jnp.where(takem, k, pk)
            v = jnp.where(takem, v, pv)
    outi_ref[:] = v


def _run_sort(keys, idx):
    R = keys.shape[0]
    return pl.pallas_call(
        _sort_body,
        out_shape=jax.ShapeDtypeStruct((R, 128), jnp.int32),
    )(keys, idx)


def _nms_body(anc_ref, del_ref, clip_ref, out_ref,
              x1_ref, y1_ref, x2_ref, y2_ref, ar_ref,
              fx1_ref, fy1_ref, fx2_ref, fy2_ref, far_ref, alive_ref, cnt_ref):
    f32 = jnp.float32
    anc = anc_ref[0]
    dlt = del_ref[0]
    ax1, ay1, ax2, ay2 = anc[0], anc[1], anc[2], anc[3]
    dx, dy, dw, dh = dlt[0], dlt[1], dlt[2], dlt[3]

    aw = ax2 - ax1 + 1.0
    ah = ay2 - ay1 + 1.0
    acx = ax1 + 0.5 * aw
    acy = ay1 + 0.5 * ah
    pcx = dx * aw + acx
    pcy = dy * ah + acy
    pw = jnp.exp(dw) * aw
    ph = jnp.exp(dh) * ah
    wm = clip_ref[0, 0:1, :]
    hm = clip_ref[0, 1:2, :]
    x1 = jnp.clip(pcx - 0.5 * pw, 0.0, wm)
    y1 = jnp.clip(pcy - 0.5 * ph, 0.0, hm)
    x2 = jnp.clip(pcx + 0.5 * pw, 0.0, wm)
    y2 = jnp.clip(pcy + 0.5 * ph, 0.0, hm)
    areas = (x2 - x1 + 1.0) * (y2 - y1 + 1.0)
    x1_ref[:] = x1
    y1_ref[:] = y1
    x2_ref[:] = x2
    y2_ref[:] = y2
    ar_ref[:] = areas
    fx1_ref[:] = x1.reshape(1, _NPAD)
    fy1_ref[:] = y1.reshape(1, _NPAD)
    fx2_ref[:] = x2.reshape(1, _NPAD)
    fy2_ref[:] = y2.reshape(1, _NPAD)
    far_ref[:] = areas.reshape(1, _NPAD)

    ii = lax.broadcasted_iota(jnp.int32, (128, 128), 0)
    jj = lax.broadcasted_iota(jnp.int32, (128, 128), 1)
    ident = (ii == jj).astype(f32)

    def tcol(rows):
        return lax.dot_general(ident, rows, (((1,), (1,)), ((), ())),
                               preferred_element_type=f32)

    tri = (ii < jj).astype(f32)

    def row_of(ref, r):
        return ref[pl.ds(r, 1), :]

    lane = lax.broadcasted_iota(jnp.int32, (_NB, 128), 1)
    rowi = lax.broadcasted_iota(jnp.int32, (_NB, 128), 0)
    gidx = (rowi * 128 + lane).astype(f32)
    alive_ref[:] = jnp.where(gidx < float(_PRE_NMS), 1.0, 0.0)
    cnt_ref[0] = 0

    def block_body(b0, carry):
        @pl.when(cnt_ref[0] < _POST_NMS)
        def _():
            x1r = row_of(x1_ref, b0)
            y1r = row_of(y1_ref, b0)
            x2r = row_of(x2_ref, b0)
            y2r = row_of(y2_ref, b0)
            arr = row_of(ar_ref, b0)
            x1c = tcol(x1r)
            y1c = tcol(y1r)
            x2c = tcol(x2r)
            y2c = tcol(y2r)
            arc = tcol(arr)

            xx1 = jnp.maximum(x1c, x1r)
            yy1 = jnp.maximum(y1c, y1r)
            xx2 = jnp.minimum(x2c, x2r)
            yy2 = jnp.minimum(y2c, y2r)
            w = jnp.maximum(0.0, xx2 - xx1 + 1.0)
            h = jnp.maximum(0.0, yy2 - yy1 + 1.0)
            inter = w * h
            iou = inter / (arc + arr - inter)
            omat = jnp.where(iou > _NMS_THRESH, tri, 0.0)

            a0 = alive_ref[pl.ds(b0, 1), :]

            def wcond(st):
                return st[1]

            def wbody(st):
                a, _ = st
                supp = jnp.dot(a, omat, preferred_element_type=f32)
                na = jnp.where(supp > 0.5, 0.0, a0)
                return na, jnp.any(na != a)

            a_fin, _ = lax.while_loop(wcond, wbody, (a0, True))
            alive_ref[pl.ds(b0, 1), :] = a_fin
            nkept = jnp.sum(a_fin)
            cnt_ref[0] = cnt_ref[0] + nkept.astype(jnp.int32)

            @pl.when(nkept > 0.0)
            def _():
                lim = (b0 * 128 + 127).astype(f32)
                lane8 = lax.broadcasted_iota(
                    jnp.int32, (1, 1024), 1).astype(f32)

                def schunk(kc, _c):
                    rs = jnp.minimum(b0 + 1 + kc * 8, _NB - 8)
                    fo = rs * 128
                    tx1 = fx1_ref[:, pl.ds(fo, 1024)]
                    ty1 = fy1_ref[:, pl.ds(fo, 1024)]
                    tx2 = fx2_ref[:, pl.ds(fo, 1024)]
                    ty2 = fy2_ref[:, pl.ds(fo, 1024)]
                    tar = far_ref[:, pl.ds(fo, 1024)]
                    sxx1 = jnp.maximum(x1c, tx1)
                    syy1 = jnp.maximum(y1c, ty1)
                    sxx2 = jnp.minimum(x2c, tx2)
                    syy2 = jnp.minimum(y2c, ty2)
                    sw = jnp.maximum(0.0, sxx2 - sxx1 + 1.0)
                    sh = jnp.maximum(0.0, syy2 - syy1 + 1.0)
                    sint = sw * sh
                    siou = sint / (arc + tar - sint)
                    smat = jnp.where(siou > _NMS_THRESH, 1.0, 0.0)
                    cntv = jnp.dot(a_fin, smat, preferred_element_type=f32)
                    later = (fo.astype(f32) + lane8) > lim
                    supp8 = jnp.where((cntv > 0.5) & later, 0.0, 1.0).reshape(8, 128)
                    rows = alive_ref[pl.ds(rs, 8), :]
                    alive_ref[pl.ds(rs, 8), :] = rows * supp8
                    return _c

                nchunks = (_NB - 1 - b0 + 7) // 8
                lax.fori_loop(0, nchunks, schunk, 0)

        return carry

    lax.fori_loop(0, _NB, block_body, 0)

    alive = alive_ref[:]
    tinc = (ii <= jj).astype(f32)
    csum_inc = jnp.dot(alive, tinc, preferred_element_type=f32)
    rs = csum_inc[:, 127:128]
    ri = lax.broadcasted_iota(jnp.int32, (_NB, _NB), 0)
    ci = lax.broadcasted_iota(jnp.int32, (_NB, _NB), 1)
    ustrict = (ci < ri).astype(f32)
    rowoff = jnp.dot(ustrict, rs, preferred_element_type=f32)
    pos = rowoff + csum_inc - alive
    posm = jnp.where(alive > 0.5, pos, 1e9)
    alive_ref[:] = posm

    srange = lax.broadcasted_iota(jnp.int32, (_POST_NMS, 1), 0).astype(f32)

    def obody(c, acc):
        prow = alive_ref[pl.ds(c, 1), :]
        sel = (prow == srange).astype(f32)
        rstack = jnp.concatenate(
            [row_of(x1_ref, c), row_of(y1_ref, c), row_of(x2_ref, c), row_of(y2_ref, c)], axis=0)
        boxc = tcol(rstack)
        return acc + jnp.dot(sel, boxc, preferred_element_type=f32)

    acc = lax.fori_loop(0, _NB, obody, jnp.zeros((_POST_NMS, 4), f32))
    out_ref[0] = acc


def _run_nms(anc4, del4, clipb):
    B = anc4.shape[0]
    return pl.pallas_call(
        _nms_body,
        grid=(B,),
        in_specs=[
            pl.BlockSpec((1, 4, _NB, 128), lambda b: (b, 0, 0, 0)),
            pl.BlockSpec((1, 4, _NB, 128), lambda b: (b, 0, 0, 0)),
            pl.BlockSpec((1, 2, 128), lambda b: (b, 0, 0)),
        ],
        out_specs=pl.BlockSpec((1, _POST_NMS, 4), lambda b: (b, 0, 0)),
        out_shape=jax.ShapeDtypeStruct((B, _POST_NMS, 4), jnp.float32),
        scratch_shapes=[
            pltpu.VMEM((_NB, 128), jnp.float32),
            pltpu.VMEM((_NB, 128), jnp.float32),
            pltpu.VMEM((_NB, 128), jnp.float32),
            pltpu.VMEM((_NB, 128), jnp.float32),
            pltpu.VMEM((_NB, 128), jnp.float32),
            pltpu.VMEM((1, _NPAD), jnp.float32),
            pltpu.VMEM((1, _NPAD), jnp.float32),
            pltpu.VMEM((1, _NPAD), jnp.float32),
            pltpu.VMEM((1, _NPAD), jnp.float32),
            pltpu.VMEM((1, _NPAD), jnp.float32),
            pltpu.VMEM((_NB, 128), jnp.float32),
            pltpu.SMEM((1,), jnp.int32),
        ],
    )(anc4, del4, clipb)


def kernel(scores, bbox_deltas, im_info):
    B = bbox_deltas.shape[0]
    anchors = jnp.asarray(_anchors_np())
    sc = scores[:, :, 1]

    kk = lax.bitcast_convert_type(sc, jnp.int32)
    kk = jnp.pad(kk, ((0, 0), (0, 65536 - kk.shape[1])), constant_values=-1)
    ii = jnp.broadcast_to(jnp.arange(65536, dtype=jnp.int32), (B, 65536))
    sorted_idx = _run_sort(kk.reshape(B * 512, 128), ii.reshape(B * 512, 128))
    oi = sorted_idx.reshape(B, 65536)[:, :_NPAD]

    anc4, del4 = _sc_gather(oi, anchors, bbox_deltas)

    wm = jnp.broadcast_to((im_info[:, 1] - 1.0)[:, None], (B, 128))
    hm = jnp.broadcast_to((im_info[:, 0] - 1.0)[:, None], (B, 128))
    clipb = jnp.stack([wm, hm], axis=1)

    boxes = _run_nms(anc4, del4, clipb)

    bcol = jnp.broadcast_to(
        jnp.arange(B, dtype=jnp.float32)[:, None, None], (B, _POST_NMS, 1))
    return jnp.concatenate([bcol, boxes], axis=2)


# speedup vs baseline: 241.7419x; 1.0471x over previous
import functools

import numpy as np
import jax
import jax.numpy as jnp
from jax import lax
from jax.experimental import pallas as pl
from jax.experimental.pallas import tpu as pltpu
from jax.experimental.pallas import tpu_sc as plsc

_FPN_SCALES = [8, 16, 32, 64, 128]
_FPN_STRIDES = [4, 8, 16, 32, 64]
_ANCHOR_RATIOS = np.array([0.5, 1.0, 2.0], dtype=np.float64)
_FEAT_SHAPES = [(128, 128), (64, 64), (32, 32), (16, 16), (8, 8)]
_PRE_NMS = 12000
_POST_NMS = 2000
_NMS_THRESH = 0.7

_NB = 94
_NPAD = _NB * 128


def _anchors_np():
    anchors = []
    for lvl in range(len(_FPN_SCALES)):
        scale, stride, shape = _FPN_SCALES[lvl], _FPN_STRIDES[lvl], _FEAT_SHAPES[lvl]
        scales_g, ratios_g = np.meshgrid(np.array([scale], dtype=np.float64), _ANCHOR_RATIOS)
        scales_f = scales_g.flatten()
        ratios_f = ratios_g.flatten()
        heights = scales_f / np.sqrt(ratios_f)
        widths = scales_f * np.sqrt(ratios_f)
        shifts_y = np.arange(0, shape[0], 1) * stride
        shifts_x = np.arange(0, shape[1], 1) * stride
        shifts_x, shifts_y = np.meshgrid(shifts_x, shifts_y)
        box_widths, box_centers_x = np.meshgrid(widths, shifts_x)
        box_heights, box_centers_y = np.meshgrid(heights, shifts_y)
        box_centers = np.stack([box_centers_x, box_centers_y], axis=2).reshape(-1, 2)
        box_sizes = np.stack([box_widths, box_heights], axis=2).reshape(-1, 2)
        anchors.append(np.concatenate(
            [box_centers - 0.5 * box_sizes, box_centers + 0.5 * box_sizes], axis=1))
    return np.concatenate(anchors, axis=0).astype(np.float32)


_roll = pltpu.roll

_GW = 32
_GCH = 6
_GPAD = 12288


def _make_sc_gather():
    mesh = plsc.VectorSubcoreMesh(core_axis_name="c", subcore_axis_name="s")

    @functools.partial(
        pl.kernel, mesh=mesh,
        out_type=jax.ShapeDtypeStruct((8, _GW, _GCH, 128), jnp.float32),
        scratch_types=[
            pltpu.VMEM((_GCH, 128), jnp.int32),
            pltpu.VMEM((_GCH, 128), jnp.int32),
            pltpu.VMEM((_GCH, 128), jnp.float32),
            pltpu.SemaphoreType.DMA,
        ],
    )
    def gk(aidx_hbm, didx_hbm, t0, t1, t2, t3, t4, t5, t6, t7,
           out_hbm, aidx_v, didx_v, buf_v, sem):
        w = lax.axis_index("s") * 2 + lax.axis_index("c")
        pltpu.sync_copy(aidx_hbm.at[w], aidx_v)
        pltpu.sync_copy(didx_hbm.at[w], didx_v)
        for c, tbl in enumerate((t0, t1, t2, t3, t4, t5, t6, t7)):
            idx_v = aidx_v if c < 4 else didx_v
            cps = [pltpu.async_copy(tbl.at[idx_v.at[j]], buf_v.at[j], sem)
                   for j in range(_GCH)]
            for cp in cps:
                cp.wait()
            pltpu.sync_copy(buf_v, out_hbm.at[c, w])

    return gk


def _sc_gather(oi, anchors, bbox_deltas):
    B, n = oi.shape
    aidx = jnp.pad(oi, ((0, 0), (0, _GPAD - n))).astype(jnp.int32)
    didx = aidx + (jnp.arange(B, dtype=jnp.int32) * 65472)[:, None]
    aidx = aidx.reshape(_GW, _GCH, 128)
    didx = didx.reshape(_GW, _GCH, 128)
    tables = [anchors[:, c] for c in range(4)]
    tables += [bbox_deltas[:, :, c].reshape(-1) for c in range(4)]
    g = _make_sc_gather()(aidx, didx, *tables)
    g = g.reshape(8, B, _GPAD // 128, 128)[:, :, :_NB, :]
    return g[0:4].transpose(1, 0, 2, 3), g[4:8].transpose(1, 0, 2, 3)


def _sort_body(key_ref, idx_ref, outi_ref):
    k = key_ref[:]
    v = idx_ref[:]
    R = k.shape[0]
    lane = lax.broadcasted_iota(jnp.int32, (R, 128), 1)
    rowl = lax.broadcasted_iota(jnp.int32, (R, 128), 0) % 512

    def partner(arr, j):
        if j < 128:
            lo = (lane & j) == 0
            return jnp.where(lo, _roll(arr, 128 - j, 1), _roll(arr, j, 1))
        m = j // 128
        a = arr.reshape(R // (2 * m), 2, m, 128)
        sw = jnp.concatenate([a[:, 1:2], a[:, 0:1]], axis=1)
        return sw.reshape(R, 128)

    for lev in range(1, 17):
        kbit = 1 << lev
        for j_exp in range(lev - 1, -1, -1):
            j = 1 << j_exp
            pk = partner(k, j)
            pv = partner(v, j)
            low = ((lane & j) == 0) if j < 128 else ((rowl & (j // 128)) == 0)
            if kbit < 128:
                desc = (lane & kbit) == 0
            else:
                desc = (rowl & (kbit // 128)) == 0
            better = (k > pk) | ((k == pk) & (v < pv))
            takem = (desc == low) == better
            k = jnp.where(takem, k, pk)
            v = jnp.where(takem, v, pv)
    outi_ref[:] = v


def _run_sort(keys, idx):
    R = keys.shape[0]
    return pl.pallas_call(
        _sort_body,
        out_shape=jax.ShapeDtypeStruct((R, 128), jnp.int32),
    )(keys, idx)


def _nms_body(anc_ref, del_ref, clip_ref, out_ref,
              x1_ref, y1_ref, x2_ref, y2_ref, ar_ref,
              fx1_ref, fy1_ref, fx2_ref, fy2_ref, far_ref, alive_ref, cnt_ref):
    f32 = jnp.float32
    anc = anc_ref[0]
    dlt = del_ref[0]
    ax1, ay1, ax2, ay2 = anc[0], anc[1], anc[2], anc[3]
    dx, dy, dw, dh = dlt[0], dlt[1], dlt[2], dlt[3]

    aw = ax2 - ax1 + 1.0
    ah = ay2 - ay1 + 1.0
    acx = ax1 + 0.5 * aw
    acy = ay1 + 0.5 * ah
    pcx = dx * aw + acx
    pcy = dy * ah + acy
    pw = jnp.exp(dw) * aw
    ph = jnp.exp(dh) * ah
    wm = clip_ref[0, 0:1, :]
    hm = clip_ref[0, 1:2, :]
    x1 = jnp.clip(pcx - 0.5 * pw, 0.0, wm)
    y1 = jnp.clip(pcy - 0.5 * ph, 0.0, hm)
    x2 = jnp.clip(pcx + 0.5 * pw, 0.0, wm)
    y2 = jnp.clip(pcy + 0.5 * ph, 0.0, hm)
    areas = (x2 - x1 + 1.0) * (y2 - y1 + 1.0)
    x1_ref[:] = x1
    y1_ref[:] = y1
    x2_ref[:] = x2
    y2_ref[:] = y2
    ar_ref[:] = areas
    fx1_ref[:] = x1.reshape(1, _NPAD)
    fy1_ref[:] = y1.reshape(1, _NPAD)
    fx2_ref[:] = x2.reshape(1, _NPAD)
    fy2_ref[:] = y2.reshape(1, _NPAD)
    far_ref[:] = areas.reshape(1, _NPAD)

    ii = lax.broadcasted_iota(jnp.int32, (128, 128), 0)
    jj = lax.broadcasted_iota(jnp.int32, (128, 128), 1)
    ident = (ii == jj).astype(f32)

    def tcol(rows):
        return lax.dot_general(ident, rows, (((1,), (1,)), ((), ())),
                               preferred_element_type=f32)

    tri = (ii < jj).astype(f32)

    def row_of(ref, r):
        return ref[pl.ds(r, 1), :]

    lane = lax.broadcasted_iota(jnp.int32, (_NB, 128), 1)
    rowi = lax.broadcasted_iota(jnp.int32, (_NB, 128), 0)
    gidx = (rowi * 128 + lane).astype(f32)
    alive_ref[:] = jnp.where(gidx < float(_PRE_NMS), 1.0, 0.0)
    cnt_ref[0] = 0

    def block_body(b0, carry):
        @pl.when(cnt_ref[0] < _POST_NMS)
        def _():
            x1r = row_of(x1_ref, b0)
            y1r = row_of(y1_ref, b0)
            x2r = row_of(x2_ref, b0)
            y2r = row_of(y2_ref, b0)
            arr = row_of(ar_ref, b0)
            x1c = tcol(x1r)
            y1c = tcol(y1r)
            x2c = tcol(x2r)
            y2c = tcol(y2r)
            arc = tcol(arr)

            xx1 = jnp.maximum(x1c, x1r)
            yy1 = jnp.maximum(y1c, y1r)
            xx2 = jnp.minimum(x2c, x2r)
            yy2 = jnp.minimum(y2c, y2r)
            w = jnp.maximum(0.0, xx2 - xx1 + 1.0)
            h = jnp.maximum(0.0, yy2 - yy1 + 1.0)
            inter = w * h
            iou = inter / (arc + arr - inter)
            omat = jnp.where(iou > _NMS_THRESH, tri, 0.0)

            a0 = alive_ref[pl.ds(b0, 1), :]

            def wcond(st):
                return st[1]

            def wbody(st):
                a, _ = st
                supp = jnp.dot(a, omat, preferred_element_type=f32)
                na = jnp.where(supp > 0.5, 0.0, a0)
                return na, jnp.any(na != a)

            a_fin, _ = lax.while_loop(wcond, wbody, (a0, True))
            alive_ref[pl.ds(b0, 1), :] = a_fin
            nkept = jnp.sum(a_fin)
            cnt_ref[0] = cnt_ref[0] + nkept.astype(jnp.int32)

            @pl.when(nkept > 0.0)
            def _():
                lim = (b0 * 128 + 127).astype(f32)
                lane8 = lax.broadcasted_iota(
                    jnp.int32, (1, 2048), 1).astype(f32)

                def schunk(kc, _c):
                    rs = jnp.minimum(b0 + 1 + kc * 16, _NB - 16)
                    fo = rs * 128
                    tx1 = fx1_ref[:, pl.ds(fo, 2048)]
                    ty1 = fy1_ref[:, pl.ds(fo, 2048)]
                    tx2 = fx2_ref[:, pl.ds(fo, 2048)]
                    ty2 = fy2_ref[:, pl.ds(fo, 2048)]
                    tar = far_ref[:, pl.ds(fo, 2048)]
                    sxx1 = jnp.maximum(x1c, tx1)
                    syy1 = jnp.maximum(y1c, ty1)
                    sxx2 = jnp.minimum(x2c, tx2)
                    syy2 = jnp.minimum(y2c, ty2)
                    sw = jnp.maximum(0.0, sxx2 - sxx1 + 1.0)
                    sh = jnp.maximum(0.0, syy2 - syy1 + 1.0)
                    sint = sw * sh
                    siou = sint / (arc + tar - sint)
                    smat = jnp.where(siou > _NMS_THRESH, 1.0, 0.0)
                    cntv = jnp.dot(a_fin, smat, preferred_element_type=f32)
                    later = (fo.astype(f32) + lane8) > lim
                    supp8 = jnp.where((cntv > 0.5) & later, 0.0, 1.0).reshape(16, 128)
                    rows = alive_ref[pl.ds(rs, 16), :]
                    alive_ref[pl.ds(rs, 16), :] = rows * supp8
                    return _c

                nchunks = (_NB - 1 - b0 + 15) // 16
                lax.fori_loop(0, nchunks, schunk, 0)

        return carry

    lax.fori_loop(0, _NB, block_body, 0)

    alive = alive_ref[:]
    tinc = (ii <= jj).astype(f32)
    csum_inc = jnp.dot(alive, tinc, preferred_element_type=f32)
    rs = csum_inc[:, 127:128]
    ri = lax.broadcasted_iota(jnp.int32, (_NB, _NB), 0)
    ci = lax.broadcasted_iota(jnp.int32, (_NB, _NB), 1)
    ustrict = (ci < ri).astype(f32)
    rowoff = jnp.dot(ustrict, rs, preferred_element_type=f32)
    pos = rowoff + csum_inc - alive
    posm = jnp.where(alive > 0.5, pos, 1e9)
    alive_ref[:] = posm

    srange = lax.broadcasted_iota(jnp.int32, (_POST_NMS, 1), 0).astype(f32)

    def obody(c, acc):
        prow = alive_ref[pl.ds(c, 1), :]
        sel = (prow == srange).astype(f32)
        rstack = jnp.concatenate(
            [row_of(x1_ref, c), row_of(y1_ref, c), row_of(x2_ref, c), row_of(y2_ref, c)], axis=0)
        boxc = tcol(rstack)
        return acc + jnp.dot(sel, boxc, preferred_element_type=f32)

    acc = lax.fori_loop(0, _NB, obody, jnp.zeros((_POST_NMS, 4), f32))
    out_ref[0] = acc


def _run_nms(anc4, del4, clipb):
    B = anc4.shape[0]
    return pl.pallas_call(
        _nms_body,
        grid=(B,),
        in_specs=[
            pl.BlockSpec((1, 4, _NB, 128), lambda b: (b, 0, 0, 0)),
            pl.BlockSpec((1, 4, _NB, 128), lambda b: (b, 0, 0, 0)),
            pl.BlockSpec((1, 2, 128), lambda b: (b, 0, 0)),
        ],
        out_specs=pl.BlockSpec((1, _POST_NMS, 4), lambda b: (b, 0, 0)),
        out_shape=jax.ShapeDtypeStruct((B, _POST_NMS, 4), jnp.float32),
        scratch_shapes=[
            pltpu.VMEM((_NB, 128), jnp.float32),
            pltpu.VMEM((_NB, 128), jnp.float32),
            pltpu.VMEM((_NB, 128), jnp.float32),
            pltpu.VMEM((_NB, 128), jnp.float32),
            pltpu.VMEM((_NB, 128), jnp.float32),
            pltpu.VMEM((1, _NPAD), jnp.float32),
            pltpu.VMEM((1, _NPAD), jnp.float32),
            pltpu.VMEM((1, _NPAD), jnp.float32),
            pltpu.VMEM((1, _NPAD), jnp.float32),
            pltpu.VMEM((1, _NPAD), jnp.float32),
            pltpu.VMEM((_NB, 128), jnp.float32),
            pltpu.SMEM((1,), jnp.int32),
        ],
    )(anc4, del4, clipb)


def kernel(scores, bbox_deltas, im_info):
    B = bbox_deltas.shape[0]
    anchors = jnp.asarray(_anchors_np())
    sc = scores[:, :, 1]

    kk = lax.bitcast_convert_type(sc, jnp.int32)
    kk = jnp.pad(kk, ((0, 0), (0, 65536 - kk.shape[1])), constant_values=-1)
    ii = jnp.broadcast_to(jnp.arange(65536, dtype=jnp.int32), (B, 65536))
    sorted_idx = _run_sort(kk.reshape(B * 512, 128), ii.reshape(B * 512, 128))
    oi = sorted_idx.reshape(B, 65536)[:, :_NPAD]

    anc4, del4 = _sc_gather(oi, anchors, bbox_deltas)

    wm = jnp.broadcast_to((im_info[:, 1] - 1.0)[:, None], (B, 128))
    hm = jnp.broadcast_to((im_info[:, 0] - 1.0)[:, None], (B, 128))
    clipb = jnp.stack([wm, hm], axis=1)

    boxes = _run_nms(anc4, del4, clipb)

    bcol = jnp.broadcast_to(
        jnp.arange(B, dtype=jnp.float32)[:, None, None], (B, _POST_NMS, 1))
    return jnp.concatenate([bcol, boxes], axis=2)
